# Initial kernel scaffold; baseline (speedup 1.0000x reference)
import jax
import jax.numpy as jnp
from jax.experimental import pallas as pl


def kernel(feat_src, edge_index, feat_edge, W_src, W_dst, b_dst, W_attn_src, W_attn_dst, W_attn_edge):
    raise NotImplementedError("write your pallas kernel here")


# speedup vs baseline: 9.7743x; 9.7743x over previous
import dataclasses
import functools

import jax
import jax.numpy as jnp
from jax import lax
from jax.experimental import pallas as pl
from jax.experimental.pallas import tpu as pltpu
from jax.experimental.pallas import tpu_sc as plsc

_N = 10000
_E = 320000
_D = 128
_DE = 16
_F = 128
_NEG = 0.2

_NTILES = 32
_EPT = _E // _NTILES
_K = 80
_NCH = _EPT // _K
_ZR = 80
_NZCH = _N // _ZR


def _tc_proj(x, Ws, Wd, b2, Wa):
    blk = 1000

    def body(x_ref, ws_ref, wd_ref, b_ref, wa_ref, fc_ref, dfc_ref, at_ref):
        xb = x_ref[...]
        fc_ref[...] = jnp.dot(xb, ws_ref[...], preferred_element_type=jnp.float32)
        dfc_ref[...] = jnp.dot(xb, wd_ref[...], preferred_element_type=jnp.float32) + b_ref[...]
        at_ref[...] = jnp.dot(xb, wa_ref[...], preferred_element_type=jnp.float32)

    return pl.pallas_call(
        body,
        grid=(_N // blk,),
        in_specs=[
            pl.BlockSpec((blk, _D), lambda i: (i, 0)),
            pl.BlockSpec((_D, _F), lambda i: (0, 0)),
            pl.BlockSpec((_D, _F), lambda i: (0, 0)),
            pl.BlockSpec((1, _F), lambda i: (0, 0)),
            pl.BlockSpec((_D, 2), lambda i: (0, 0)),
        ],
        out_specs=[
            pl.BlockSpec((blk, _F), lambda i: (i, 0)),
            pl.BlockSpec((blk, _F), lambda i: (i, 0)),
            pl.BlockSpec((blk, 2), lambda i: (i, 0)),
        ],
        out_shape=[
            jax.ShapeDtypeStruct((_N, _F), jnp.float32),
            jax.ShapeDtypeStruct((_N, _F), jnp.float32),
            jax.ShapeDtypeStruct((_N, 2), jnp.float32),
        ],
    )(x, Ws, Wd, b2, Wa)


def _tc_edge(fe3, w2):
    nb, blk = fe3.shape[0], fe3.shape[1]

    def body(x_ref, w_ref, o_ref):
        o_ref[0, 0, :] = jnp.sum(x_ref[0] * w_ref[...], axis=1)

    return pl.pallas_call(
        body,
        grid=(nb,),
        in_specs=[
            pl.BlockSpec((1, blk, _DE), lambda i: (i, 0, 0)),
            pl.BlockSpec((1, _DE), lambda i: (0, 0)),
        ],
        out_specs=pl.BlockSpec((1, 1, blk), lambda i: (i, 0, 0)),
        out_shape=jax.ShapeDtypeStruct((nb, 1, blk), jnp.float32),
    )(fe3, w2)


def _tc_final(partials, dfc):
    blk = 1000

    def body(p0_ref, p1_ref, d_ref, o_ref):
        o_ref[...] = p0_ref[0] + p1_ref[0] + d_ref[...]

    return pl.pallas_call(
        body,
        grid=(_N // blk,),
        in_specs=[
            pl.BlockSpec((1, blk, _F), lambda i: (0, i, 0)),
            pl.BlockSpec((1, blk, _F), lambda i: (1, i, 0)),
            pl.BlockSpec((blk, _F), lambda i: (i, 0)),
        ],
        out_specs=pl.BlockSpec((blk, _F), lambda i: (i, 0)),
        out_shape=jax.ShapeDtypeStruct((_N, _F), jnp.float32),
    )(partials, partials, dfc)


def _sc_aggregate(edge_index, ae, tab, feat_fc):
    mesh = plsc.VectorSubcoreMesh(core_axis_name="c", subcore_axis_name="s")
    cp = pltpu.CompilerParams()
    if "needs_layout_passes" in pltpu.CompilerParams.__dataclass_fields__:
        cp = dataclasses.replace(cp, needs_layout_passes=False)

    @functools.partial(
        pl.kernel,
        out_type=jax.ShapeDtypeStruct((2, _N, _F), jnp.float32),
        mesh=mesh,
        scratch_types=[
            pltpu.VMEM((_K,), jnp.int32),
            pltpu.VMEM((_K,), jnp.int32),
            pltpu.VMEM((_K,), jnp.float32),
            pltpu.VMEM((_K, _F), jnp.float32),
            pltpu.VMEM((2 * _N,), jnp.float32),
            pltpu.VMEM_SHARED((_N, _F), jnp.float32),
            pltpu.SemaphoreType.DMA,
        ],
        compiler_params=cp,
    )
    def sc_kernel(ei_hbm, ae_hbm, tab_hbm, fc_hbm, out_hbm,
                  sidx_v, didx_v, a_v, rows_v, tab_v, acc_sh, sem):
        c = lax.axis_index("c")
        s = lax.axis_index("s")
        gid = c * 16 + s

        pltpu.sync_copy(tab_hbm, tab_v)

        @pl.loop(0, _ZR)
        def _(i):
            for r in range(_F // 16):
                rows_v[i, pl.ds(r * 16, 16)] = jnp.zeros((16,), jnp.float32)

        @pl.loop(s, _NZCH, step=16)
        def _(g):
            row = pl.multiple_of(g * _ZR, 8)
            pltpu.sync_copy(rows_v, acc_sh.at[pl.ds(row, _ZR)])

        plsc.subcore_barrier()

        tile_base = gid * _EPT

        @pl.loop(0, _NCH)
        def _(j):
            base = pl.multiple_of(tile_base + j * _K, 8)
            pltpu.sync_copy(ei_hbm.at[pl.ds(base, _K)], sidx_v)
            pltpu.sync_copy(ei_hbm.at[pl.ds(_E + base, _K)], didx_v)
            pltpu.sync_copy(ae_hbm.at[pl.ds(base, _K)], a_v)
            gat = pltpu.async_copy(fc_hbm.at[sidx_v], rows_v, sem)
            for v in range(_K // 16):
                sl = pl.ds(v * 16, 16)
                e = (plsc.load_gather(tab_v, [sidx_v[sl] * 2])
                     + plsc.load_gather(tab_v, [didx_v[sl] * 2 + 1])
                     + a_v[sl])
                a_v[sl] = jnp.maximum(e, 0.0) + _NEG * jnp.minimum(e, 0.0)
            gat.wait()

            @pl.loop(0, _K)
            def _(k):
                ak = plsc.load_gather(a_v, [jnp.zeros((16,), jnp.int32) + k])
                for r in range(_F // 16):
                    sl = pl.ds(r * 16, 16)
                    rows_v[k, sl] = rows_v[k, sl] * ak

            pltpu.sync_copy(rows_v, acc_sh.at[didx_v], add=True)

        plsc.subcore_barrier()

        @pl.loop(s, _NZCH, step=16)
        def _(g):
            row = pl.multiple_of(g * _ZR, 8)
            pltpu.sync_copy(acc_sh.at[pl.ds(row, _ZR)],
                            out_hbm.at[c, pl.ds(row, _ZR)])

    return sc_kernel(edge_index, ae, tab, feat_fc)


def kernel(feat_src, edge_index, feat_edge, W_src, W_dst, b_dst,
           W_attn_src, W_attn_dst, W_attn_edge):
    W_attn = jnp.concatenate([W_attn_src, W_attn_dst], axis=1)
    feat_fc, dst_fc, attn2 = _tc_proj(
        feat_src, W_src, W_dst, b_dst.reshape(1, _F), W_attn)
    ae = _tc_edge(feat_edge.reshape(_E // 3200, 3200, _DE),
                  W_attn_edge.reshape(1, _DE)).reshape(_E)
    tab = attn2.reshape(2 * _N)
    partials = _sc_aggregate(edge_index.reshape(2 * _E), ae, tab, feat_fc)
    out = _tc_final(partials, dst_fc)
    return out.reshape(_N, 1, _F)


# speedup vs baseline: 16.4047x; 1.6784x over previous
import dataclasses
import functools

import jax
import jax.numpy as jnp
from jax import lax
from jax.experimental import pallas as pl
from jax.experimental.pallas import tpu as pltpu
from jax.experimental.pallas import tpu_sc as plsc

_N = 10000
_E = 320000
_D = 128
_DE = 16
_F = 128
_NEG = 0.2

_NTILES = 32
_EPT = _E // _NTILES
_K = 80
_NCH = _EPT // _K
_ZR = 80
_NZCH = _N // _ZR
_SBC = 25
_SBE = _SBC * _K
_NSB = _EPT // _SBE


def _tc_proj(x, Ws, Wd, b2, Wa):
    blk = 1000

    def body(x_ref, ws_ref, wd_ref, b_ref, wa_ref, fc_ref, dfc_ref, at_ref):
        xb = x_ref[...]
        fc_ref[...] = jnp.dot(xb, ws_ref[...], preferred_element_type=jnp.float32)
        dfc_ref[...] = jnp.dot(xb, wd_ref[...], preferred_element_type=jnp.float32) + b_ref[...]
        at_ref[...] = jnp.dot(xb, wa_ref[...], preferred_element_type=jnp.float32)

    return pl.pallas_call(
        body,
        grid=(_N // blk,),
        in_specs=[
            pl.BlockSpec((blk, _D), lambda i: (i, 0)),
            pl.BlockSpec((_D, _F), lambda i: (0, 0)),
            pl.BlockSpec((_D, _F), lambda i: (0, 0)),
            pl.BlockSpec((1, _F), lambda i: (0, 0)),
            pl.BlockSpec((_D, 2), lambda i: (0, 0)),
        ],
        out_specs=[
            pl.BlockSpec((blk, _F), lambda i: (i, 0)),
            pl.BlockSpec((blk, _F), lambda i: (i, 0)),
            pl.BlockSpec((blk, 2), lambda i: (i, 0)),
        ],
        out_shape=[
            jax.ShapeDtypeStruct((_N, _F), jnp.float32),
            jax.ShapeDtypeStruct((_N, _F), jnp.float32),
            jax.ShapeDtypeStruct((_N, 2), jnp.float32),
        ],
    )(x, Ws, Wd, b2, Wa)


def _tc_edge(fe3, w2):
    nb, blk = fe3.shape[0], fe3.shape[1]

    def body(x_ref, w_ref, o_ref):
        o_ref[0, 0, :] = jnp.sum(x_ref[0] * w_ref[...], axis=1)

    return pl.pallas_call(
        body,
        grid=(nb,),
        in_specs=[
            pl.BlockSpec((1, blk, _DE), lambda i: (i, 0, 0)),
            pl.BlockSpec((1, _DE), lambda i: (0, 0)),
        ],
        out_specs=pl.BlockSpec((1, 1, blk), lambda i: (i, 0, 0)),
        out_shape=jax.ShapeDtypeStruct((nb, 1, blk), jnp.float32),
    )(fe3, w2)


def _tc_final(partials, dfc):
    blk = 1000

    def body(p0_ref, p1_ref, d_ref, o_ref):
        o_ref[...] = p0_ref[0] + p1_ref[0] + d_ref[...]

    return pl.pallas_call(
        body,
        grid=(_N // blk,),
        in_specs=[
            pl.BlockSpec((1, blk, _F), lambda i: (0, i, 0)),
            pl.BlockSpec((1, blk, _F), lambda i: (1, i, 0)),
            pl.BlockSpec((blk, _F), lambda i: (i, 0)),
        ],
        out_specs=pl.BlockSpec((blk, _F), lambda i: (i, 0)),
        out_shape=jax.ShapeDtypeStruct((_N, _F), jnp.float32),
    )(partials, partials, dfc)


def _sc_aggregate(edge_index, ae, tab, feat_fc):
    mesh = plsc.VectorSubcoreMesh(core_axis_name="c", subcore_axis_name="s")
    cp = pltpu.CompilerParams()
    if "needs_layout_passes" in pltpu.CompilerParams.__dataclass_fields__:
        cp = dataclasses.replace(cp, needs_layout_passes=False)

    @functools.partial(
        pl.kernel,
        out_type=jax.ShapeDtypeStruct((2, _N, _F), jnp.float32),
        mesh=mesh,
        scratch_types=[
            pltpu.VMEM((_SBE,), jnp.int32),
            pltpu.VMEM((_SBE,), jnp.int32),
            pltpu.VMEM((_SBE,), jnp.float32),
            pltpu.VMEM((_K,), jnp.int32),
            pltpu.VMEM((_K,), jnp.int32),
            pltpu.VMEM((_K,), jnp.float32),
            pltpu.VMEM((_K, _F), jnp.float32),
            pltpu.VMEM((_K, _F), jnp.float32),
            pltpu.VMEM((2 * _N,), jnp.float32),
            pltpu.VMEM_SHARED((_N, _F), jnp.float32),
            pltpu.SemaphoreType.DMA,
        ],
        compiler_params=cp,
    )
    def sc_kernel(ei_hbm, ae_hbm, tab_hbm, fc_hbm, out_hbm,
                  sidx_v, didx_v, ae_v, dA_v, dB_v, a_v, rowsA_v, rowsB_v,
                  tab_v, acc_sh, sem):
        c = lax.axis_index("c")
        s = lax.axis_index("s")
        gid = c * 16 + s
        tile_base = gid * _EPT

        pltpu.sync_copy(tab_hbm, tab_v)

        @pl.loop(0, _ZR)
        def _(i):
            for r in range(_F // 16):
                rowsA_v[i, pl.ds(r * 16, 16)] = jnp.zeros((16,), jnp.float32)

        @pl.loop(s, _NZCH, step=16)
        def _(g):
            row = pl.multiple_of(g * _ZR, 8)
            pltpu.sync_copy(rowsA_v, acc_sh.at[pl.ds(row, _ZR)])

        plsc.subcore_barrier()

        def fire_gather(j, rows_buf):
            return pltpu.async_copy(
                fc_hbm.at[sidx_v.at[pl.ds(j * _K, _K)]], rows_buf, sem)

        def wait_gather(rows_buf):
            pltpu.make_async_copy(
                fc_hbm.at[pl.ds(0, _K)], rows_buf, sem).wait()

        def process(j, d_buf, rows_buf):
            for v in range(_K // 16):
                sl = pl.ds(v * 16, 16)
                gl = pl.ds(j * _K + v * 16, 16)
                d16 = didx_v[gl]
                d_buf[sl] = d16
                e = (plsc.load_gather(tab_v, [sidx_v[gl] * 2])
                     + plsc.load_gather(tab_v, [d16 * 2 + 1])
                     + ae_v[gl])
                a_v[sl] = jnp.maximum(e, 0.0) + _NEG * jnp.minimum(e, 0.0)
            wait_gather(rows_buf)

            @pl.loop(0, _K, step=4)
            def _(k0):
                for u in range(4):
                    k = k0 + u
                    ak = plsc.load_gather(a_v, [jnp.zeros((16,), jnp.int32) + k])
                    for r in range(_F // 16):
                        sl = pl.ds(r * 16, 16)
                        rows_buf[k, sl] = rows_buf[k, sl] * ak

            pltpu.sync_copy(rows_buf, acc_sh.at[d_buf], add=True)

        @pl.loop(0, _NSB)
        def _(sb):
            ebase = pl.multiple_of(tile_base + sb * _SBE, 8)
            pltpu.sync_copy(ei_hbm.at[pl.ds(ebase, _SBE)], sidx_v)
            pltpu.sync_copy(ei_hbm.at[pl.ds(_E + ebase, _SBE)], didx_v)
            pltpu.sync_copy(ae_hbm.at[pl.ds(ebase, _SBE)], ae_v)
            fire_gather(0, rowsA_v)

            @pl.loop(0, _SBC - 1, step=2)
            def _(j):
                fire_gather(j + 1, rowsB_v)
                process(j, dA_v, rowsA_v)
                fire_gather(j + 2, rowsA_v)
                process(j + 1, dB_v, rowsB_v)

            process(_SBC - 1, dA_v, rowsA_v)

        plsc.subcore_barrier()

        @pl.loop(s, _NZCH, step=16)
        def _(g):
            row = pl.multiple_of(g * _ZR, 8)
            pltpu.sync_copy(acc_sh.at[pl.ds(row, _ZR)],
                            out_hbm.at[c, pl.ds(row, _ZR)])

    return sc_kernel(edge_index, ae, tab, feat_fc)


def kernel(feat_src, edge_index, feat_edge, W_src, W_dst, b_dst,
           W_attn_src, W_attn_dst, W_attn_edge):
    W_attn = jnp.concatenate([W_attn_src, W_attn_dst], axis=1)
    feat_fc, dst_fc, attn2 = _tc_proj(
        feat_src, W_src, W_dst, b_dst.reshape(1, _F), W_attn)
    ae = _tc_edge(feat_edge.reshape(_E // 3200, 3200, _DE),
                  W_attn_edge.reshape(1, _DE)).reshape(_E)
    tab = attn2.reshape(2 * _N)
    partials = _sc_aggregate(edge_index.reshape(2 * _E), ae, tab, feat_fc)
    out = _tc_final(partials, dst_fc)
    return out.reshape(_N, 1, _F)
